# baseline (speedup 1.0000x reference)
import functools

import jax
import jax.numpy as jnp
from jax import lax
from jax.experimental import pallas as pl
from jax.experimental.pallas import tpu as pltpu
from jax.experimental.pallas import tpu_sc as plsc

EMBED = 16


@functools.lru_cache(maxsize=None)
def _build_transpose(vocab):
    info = plsc.get_sparse_core_info()
    nw = info.num_cores * info.num_subcores
    blk = 512
    n_full = vocab // blk
    tail = vocab - n_full * blk
    n_pairs = (n_full + 2 * nw - 1) // (2 * nw)

    mesh = plsc.VectorSubcoreMesh(core_axis_name="c", subcore_axis_name="s")

    n_ct = blk // 128
    scratch = [
        pltpu.VMEM((2, n_ct, EMBED, 128), jnp.float32),
        pltpu.VMEM((blk * EMBED,), jnp.float32),
        pltpu.VMEM((blk * EMBED,), jnp.float32),
    ] + [pltpu.SemaphoreType.DMA] * 4

    @functools.partial(
        pl.kernel,
        out_type=jax.ShapeDtypeStruct((vocab * EMBED,), jnp.float32),
        mesh=mesh,
        scratch_types=scratch,
        compiler_params=pltpu.CompilerParams(
            use_tc_tiling_on_sc=True,
            needs_layout_passes=False,
            disable_bounds_checks=True,
        ),
    )
    def tr_kernel(tab_hbm, tail_hbm, out_hbm, buf, trans0, trans1, *sems):
        trans = (trans0, trans1)
        i_sems = sems[:2]
        o_sems = sems[2:]
        wid = lax.axis_index("s") * info.num_cores + lax.axis_index("c")
        lanes = lax.iota(jnp.int32, EMBED)

        def in_descs(b, s, width):
            v0 = b * blk
            descs = []
            for c in range(width // 128):
                for eh in range(2):
                    descs.append(pltpu.make_async_copy(
                        tab_hbm.at[pl.ds(8 * eh, 8), pl.ds(v0 + 128 * c, 128)],
                        buf.at[s, c, pl.ds(8 * eh, 8), :], i_sems[s],
                    ))
            return descs

        scatter_idx = [lanes * EMBED + e for e in range(EMBED)]

        def transpose(s):
            @plsc.parallel_loop(0, 8, unroll=4)
            def body(jg):
                jbase = jg * (16 * EMBED)
                for ct in range(n_ct):
                    cbase = jbase + ct * (128 * EMBED)
                    for e in range(EMBED):
                        v = buf[s, ct, e, pl.ds(jg * 16, 16)]
                        plsc.store_scatter(
                            trans[s], [scatter_idx[e] + cbase], v
                        )

        def out_desc(b, s):
            return pltpu.make_async_copy(
                trans[s], out_hbm.at[pl.ds(b * blk * EMBED, blk * EMBED)],
                o_sems[s],
            )

        def drain_out(s):
            pltpu.make_async_copy(
                out_hbm.at[pl.ds(0, blk * EMBED)], trans[s], o_sems[s]
            ).wait()

        def pair_body(k, _):
            for s in range(2):
                b = wid + nw * (2 * k + s)
                descs = in_descs(b, s, blk)

                @pl.when(b < n_full)
                def _():
                    for d in descs:
                        d.start()

            for s in range(2):
                b = wid + nw * (2 * k + s)
                descs = in_descs(b, s, blk)
                oc = out_desc(b, s)

                @pl.when(b < n_full)
                def _():
                    for d in descs:
                        d.wait()

                    @pl.when(2 * k + s >= 2)
                    def _():
                        drain_out(s)

                    transpose(s)
                    oc.start()

            return 0

        lax.fori_loop(0, n_pairs, pair_body, 0)
        n_mine = (n_full - 1 - wid) // nw + 1
        for s in range(2):

            @pl.when(n_mine >= s + 1)
            def _(s=s):
                drain_out(s)

        if tail:
            v0 = n_full * blk
            ic_t = pltpu.make_async_copy(
                tail_hbm, trans[0].at[pl.ds(0, tail * EMBED)], i_sems[0]
            )
            oc_t = pltpu.make_async_copy(
                trans[0].at[pl.ds(0, tail * EMBED)],
                out_hbm.at[pl.ds(v0 * EMBED, tail * EMBED)], o_sems[0],
            )

            @pl.when(wid == nw - 1)
            def _():
                ic_t.start()
                ic_t.wait()
                oc_t.start()
                oc_t.wait()

    return tr_kernel


@functools.lru_cache(maxsize=None)
def _build(hist, batch, vocab):
    info = plsc.get_sparse_core_info()
    nw = info.num_cores * info.num_subcores
    bq = 1024
    n_bq = batch // bq
    n_tasks_total = hist * n_bq
    n_tasks = n_tasks_total // nw

    mesh = plsc.VectorSubcoreMesh(core_axis_name="c", subcore_axis_name="s")

    scratch = [
        pltpu.VMEM((n_tasks, bq), jnp.int32),
        pltpu.VMEM((2, bq, EMBED), jnp.float32),
        pltpu.VMEM((2, EMBED, bq), jnp.float32),
    ] + [pltpu.SemaphoreType.DMA] * 5

    @functools.partial(
        pl.kernel,
        out_type=jax.ShapeDtypeStruct((hist, EMBED, batch), jnp.float32),
        mesh=mesh,
        scratch_types=scratch,
        compiler_params=pltpu.CompilerParams(
            use_tc_tiling_on_sc=False,
            needs_layout_passes=False,
            disable_bounds_checks=True,
        ),
    )
    def emb_kernel(idx_hbm, table_hbm, out_hbm, idxv, rows, outb, *sems):
        g_sems = sems[:2]
        o_sems = sems[2:4]
        i_sem = sems[4]
        wid = lax.axis_index("s") * info.num_cores + lax.axis_index("c")
        t_base = wid * n_tasks

        def task_hb(k):
            t = t_base + k
            return t // n_bq, (t % n_bq) * bq

        idx_cps = []
        for k in range(n_tasks):
            h, b0 = task_hb(k)
            idx_cps.append(pltpu.async_copy(
                idx_hbm.at[h, pl.ds(b0, bq)], idxv.at[k], i_sem
            ))
        for cp in idx_cps:
            cp.wait()

        def gather_start(k, s):
            return pltpu.async_copy(
                table_hbm.at[idxv.at[k]], rows.at[s], g_sems[s]
            )

        def out_start(k, s):
            h, b0 = task_hb(k)
            return pltpu.async_copy(
                outb.at[s], out_hbm.at[h, :, pl.ds(b0, bq)], o_sems[s]
            )

        lanes = lax.iota(jnp.int32, EMBED)
        col_idx = [jnp.full((EMBED,), e, jnp.int32) for e in range(EMBED)]

        def transpose(s):
            @plsc.parallel_loop(0, bq // EMBED, unroll=4)
            def body(j):
                row_idx = lanes + j * EMBED
                for e in range(EMBED):
                    v = plsc.load_gather(rows.at[s], [row_idx, col_idx[e]])
                    outb[s, e, pl.ds(j * EMBED, EMBED)] = v

        g_cp = [None, None]
        o_cp = [None, None]
        for t in range(n_tasks):
            s = t % 2
            g_cp[s] = gather_start(t, s)
            if t > 0:
                sp = 1 - s
                g_cp[sp].wait()
                if o_cp[sp] is not None:
                    o_cp[sp].wait()
                transpose(sp)
                o_cp[sp] = out_start(t - 1, sp)
        s = (n_tasks - 1) % 2
        g_cp[s].wait()
        if o_cp[s] is not None:
            o_cp[s].wait()
        transpose(s)
        o_cp[s] = out_start(n_tasks - 1, s)
        for s in range(2):
            if o_cp[s] is not None:
                o_cp[s].wait()

    return emb_kernel


def kernel(x, table):
    batch, _, hist = x.shape
    vocab = table.shape[0]
    idx_t = x[:, 0, :].T
    tail_flat = table[(vocab // 512) * 512:, :].reshape(-1)
    table_rm = _build_transpose(vocab)(table.T, tail_flat).reshape(vocab, EMBED)
    fn = _build(hist, batch, vocab)
    out = fn(idx_t, table_rm)
    return jnp.transpose(out, (2, 0, 1))


# speedup vs baseline: 1.0139x; 1.0139x over previous
import functools

import jax
import jax.numpy as jnp
from jax import lax
from jax.experimental import pallas as pl
from jax.experimental.pallas import tpu as pltpu
from jax.experimental.pallas import tpu_sc as plsc

EMBED = 16


@functools.lru_cache(maxsize=None)
def _build_transpose(vocab):
    info = plsc.get_sparse_core_info()
    nw = info.num_cores * info.num_subcores
    blk = 512
    n_full = vocab // blk
    tail = vocab - n_full * blk
    n_pairs = (n_full + 2 * nw - 1) // (2 * nw)

    mesh = plsc.VectorSubcoreMesh(core_axis_name="c", subcore_axis_name="s")

    n_ct = blk // 128
    scratch = [
        pltpu.VMEM((2, n_ct, EMBED, 128), jnp.float32),
        pltpu.VMEM((blk * EMBED,), jnp.float32),
        pltpu.VMEM((blk * EMBED,), jnp.float32),
    ] + [pltpu.SemaphoreType.DMA] * 4

    @functools.partial(
        pl.kernel,
        out_type=jax.ShapeDtypeStruct((vocab * EMBED,), jnp.float32),
        mesh=mesh,
        scratch_types=scratch,
        compiler_params=pltpu.CompilerParams(
            use_tc_tiling_on_sc=True,
            needs_layout_passes=False,
            disable_bounds_checks=True,
        ),
    )
    def tr_kernel(tab_hbm, tail_hbm, out_hbm, buf, trans0, trans1, *sems):
        trans = (trans0, trans1)
        i_sems = sems[:2]
        o_sems = sems[2:]
        wid = lax.axis_index("s") * info.num_cores + lax.axis_index("c")
        lanes = lax.iota(jnp.int32, EMBED)

        def in_descs(b, s, width):
            v0 = b * blk
            descs = []
            for c in range(width // 128):
                for eh in range(2):
                    descs.append(pltpu.make_async_copy(
                        tab_hbm.at[pl.ds(8 * eh, 8), pl.ds(v0 + 128 * c, 128)],
                        buf.at[s, c, pl.ds(8 * eh, 8), :], i_sems[s],
                    ))
            return descs

        scatter_idx = [lanes * EMBED + e for e in range(EMBED)]

        def transpose(s):
            @plsc.parallel_loop(0, 8, unroll=2)
            def body(jg):
                jbase = jg * (16 * EMBED)
                for ct in range(n_ct):
                    cbase = jbase + ct * (128 * EMBED)
                    for e in range(EMBED):
                        v = buf[s, ct, e, pl.ds(jg * 16, 16)]
                        plsc.store_scatter(
                            trans[s], [scatter_idx[e] + cbase], v
                        )

        def out_desc(b, s):
            return pltpu.make_async_copy(
                trans[s], out_hbm.at[pl.ds(b * blk * EMBED, blk * EMBED)],
                o_sems[s],
            )

        def drain_out(s):
            pltpu.make_async_copy(
                out_hbm.at[pl.ds(0, blk * EMBED)], trans[s], o_sems[s]
            ).wait()

        def pair_body(k, _):
            for s in range(2):
                b = wid + nw * (2 * k + s)
                descs = in_descs(b, s, blk)

                @pl.when(b < n_full)
                def _():
                    for d in descs:
                        d.start()

            for s in range(2):
                b = wid + nw * (2 * k + s)
                descs = in_descs(b, s, blk)
                oc = out_desc(b, s)

                @pl.when(b < n_full)
                def _():
                    for d in descs:
                        d.wait()

                    @pl.when(2 * k + s >= 2)
                    def _():
                        drain_out(s)

                    transpose(s)
                    oc.start()

            return 0

        lax.fori_loop(0, n_pairs, pair_body, 0)
        n_mine = (n_full - 1 - wid) // nw + 1
        for s in range(2):

            @pl.when(n_mine >= s + 1)
            def _(s=s):
                drain_out(s)

        if tail:
            v0 = n_full * blk
            ic_t = pltpu.make_async_copy(
                tail_hbm, trans[0].at[pl.ds(0, tail * EMBED)], i_sems[0]
            )
            oc_t = pltpu.make_async_copy(
                trans[0].at[pl.ds(0, tail * EMBED)],
                out_hbm.at[pl.ds(v0 * EMBED, tail * EMBED)], o_sems[0],
            )

            @pl.when(wid == nw - 1)
            def _():
                ic_t.start()
                ic_t.wait()
                oc_t.start()
                oc_t.wait()

    return tr_kernel


@functools.lru_cache(maxsize=None)
def _build(hist, batch, vocab):
    info = plsc.get_sparse_core_info()
    nw = info.num_cores * info.num_subcores
    bq = 1024
    n_bq = batch // bq
    n_tasks_total = hist * n_bq
    n_tasks = n_tasks_total // nw

    mesh = plsc.VectorSubcoreMesh(core_axis_name="c", subcore_axis_name="s")

    scratch = [
        pltpu.VMEM((n_tasks, bq), jnp.int32),
        pltpu.VMEM((3, bq, EMBED), jnp.float32),
        pltpu.VMEM((2, EMBED, bq), jnp.float32),
    ] + [pltpu.SemaphoreType.DMA] * 6

    @functools.partial(
        pl.kernel,
        out_type=jax.ShapeDtypeStruct((hist, EMBED, batch), jnp.float32),
        mesh=mesh,
        scratch_types=scratch,
        compiler_params=pltpu.CompilerParams(
            use_tc_tiling_on_sc=False,
            needs_layout_passes=False,
            disable_bounds_checks=True,
        ),
    )
    def emb_kernel(idx_hbm, table_hbm, out_hbm, idxv, rows, outb, *sems):
        g_sems = sems[:3]
        o_sems = sems[3:5]
        i_sem = sems[5]
        wid = lax.axis_index("s") * info.num_cores + lax.axis_index("c")
        t_base = wid * n_tasks

        def task_hb(k):
            t = t_base + k
            return t // n_bq, (t % n_bq) * bq

        idx_cps = []
        for k in range(n_tasks):
            h, b0 = task_hb(k)
            idx_cps.append(pltpu.async_copy(
                idx_hbm.at[h, pl.ds(b0, bq)], idxv.at[k], i_sem
            ))
        for cp in idx_cps:
            cp.wait()

        def gather_start(k, s):
            return pltpu.async_copy(
                table_hbm.at[idxv.at[k]], rows.at[s], g_sems[s]
            )

        def out_start(k, s):
            h, b0 = task_hb(k)
            return pltpu.async_copy(
                outb.at[s], out_hbm.at[h, :, pl.ds(b0, bq)], o_sems[s]
            )

        lanes = lax.iota(jnp.int32, EMBED)
        col_idx = [jnp.full((EMBED,), e, jnp.int32) for e in range(EMBED)]

        def transpose(sr, so):
            @plsc.parallel_loop(0, bq // EMBED, unroll=4)
            def body(j):
                row_idx = lanes + j * EMBED
                for e in range(EMBED):
                    v = plsc.load_gather(rows.at[sr], [row_idx, col_idx[e]])
                    outb[so, e, pl.ds(j * EMBED, EMBED)] = v

        def finish(tp):
            srp, so = tp % 3, tp % 2
            g_cp[srp].wait()
            if o_cp[so] is not None:
                o_cp[so].wait()
            transpose(srp, so)
            o_cp[so] = out_start(tp, so)

        g_cp = [None, None, None]
        o_cp = [None, None]
        for t in range(n_tasks):
            g_cp[t % 3] = gather_start(t, t % 3)
            if t > 0:
                finish(t - 1)
        finish(n_tasks - 1)
        for s in range(2):
            if o_cp[s] is not None:
                o_cp[s].wait()

    return emb_kernel


def kernel(x, table):
    batch, _, hist = x.shape
    vocab = table.shape[0]
    idx_t = x[:, 0, :].T
    tail_flat = table[(vocab // 512) * 512:, :].reshape(-1)
    table_rm = _build_transpose(vocab)(table.T, tail_flat).reshape(vocab, EMBED)
    fn = _build(hist, batch, vocab)
    out = fn(idx_t, table_rm)
    return jnp.transpose(out, (2, 0, 1))


# speedup vs baseline: 1.0777x; 1.0629x over previous
import functools

import jax
import jax.numpy as jnp
from jax import lax
from jax.experimental import pallas as pl
from jax.experimental.pallas import tpu as pltpu
from jax.experimental.pallas import tpu_sc as plsc

EMBED = 16


@functools.lru_cache(maxsize=None)
def _build_transpose(vocab):
    info = plsc.get_sparse_core_info()
    nw = info.num_cores * info.num_subcores
    blk = 512
    n_full = vocab // blk
    tail = vocab - n_full * blk
    n_pairs = (n_full + 2 * nw - 1) // (2 * nw)

    mesh = plsc.VectorSubcoreMesh(core_axis_name="c", subcore_axis_name="s")

    n_ct = blk // 128
    scratch = [
        pltpu.VMEM((4, n_ct, EMBED, 128), jnp.float32),
        pltpu.VMEM((blk * EMBED,), jnp.float32),
        pltpu.VMEM((blk * EMBED,), jnp.float32),
    ] + [pltpu.SemaphoreType.DMA] * 6

    @functools.partial(
        pl.kernel,
        out_type=jax.ShapeDtypeStruct((vocab * EMBED,), jnp.float32),
        mesh=mesh,
        scratch_types=scratch,
        compiler_params=pltpu.CompilerParams(
            use_tc_tiling_on_sc=True,
            needs_layout_passes=False,
            disable_bounds_checks=True,
        ),
    )
    def tr_kernel(tab_hbm, tail_hbm, out_hbm, buf, trans0, trans1, *sems):
        trans = (trans0, trans1)
        i_sems = sems[:4]
        o_sems = sems[4:]
        wid = lax.axis_index("s") * info.num_cores + lax.axis_index("c")
        lanes = lax.iota(jnp.int32, EMBED)

        def in_descs(b, s, width):
            v0 = b * blk
            descs = []
            for c in range(width // 128):
                for eh in range(2):
                    descs.append(pltpu.make_async_copy(
                        tab_hbm.at[pl.ds(8 * eh, 8), pl.ds(v0 + 128 * c, 128)],
                        buf.at[s, c, pl.ds(8 * eh, 8), :], i_sems[s],
                    ))
            return descs

        scatter_idx = [lanes * EMBED + e for e in range(EMBED)]

        def transpose(bs, ts):
            @plsc.parallel_loop(0, 8, unroll=2)
            def body(jg):
                jbase = jg * (16 * EMBED)
                for ct in range(n_ct):
                    cbase = jbase + ct * (128 * EMBED)
                    for e in range(EMBED):
                        v = buf[bs, ct, e, pl.ds(jg * 16, 16)]
                        plsc.store_scatter(
                            trans[ts], [scatter_idx[e] + cbase], v
                        )

        def out_desc(b, s):
            return pltpu.make_async_copy(
                trans[s], out_hbm.at[pl.ds(b * blk * EMBED, blk * EMBED)],
                o_sems[s],
            )

        def drain_out(s):
            pltpu.make_async_copy(
                out_hbm.at[pl.ds(0, blk * EMBED)], trans[s], o_sems[s]
            ).wait()

        def start_pair(kp, base_slot):
            for s in range(2):
                b = wid + nw * (2 * kp + s)
                descs = in_descs(b, base_slot + s, blk)

                @pl.when(b < n_full)
                def _():
                    for d in descs:
                        d.start()

        def finish_pair(k, base_slot):
            for s in range(2):
                b = wid + nw * (2 * k + s)
                descs = in_descs(b, base_slot + s, blk)
                oc = out_desc(b, s)

                @pl.when(b < n_full)
                def _():
                    for d in descs:
                        d.wait()

                    @pl.when(2 * k + s >= 2)
                    def _():
                        drain_out(s)

                    transpose(base_slot + s, s)
                    oc.start()

        def pair_body(k, _):
            for par in range(2):

                @pl.when(k % 2 == par)
                def _(par=par):
                    start_pair(k + 1, 2 * (1 - par))
                    finish_pair(k, 2 * par)

            return 0

        start_pair(0, 0)
        lax.fori_loop(0, n_pairs, pair_body, 0)
        n_mine = (n_full - 1 - wid) // nw + 1
        for s in range(2):

            @pl.when(n_mine >= s + 1)
            def _(s=s):
                drain_out(s)

        if tail:
            v0 = n_full * blk
            ic_t = pltpu.make_async_copy(
                tail_hbm, trans[0].at[pl.ds(0, tail * EMBED)], i_sems[0]
            )
            oc_t = pltpu.make_async_copy(
                trans[0].at[pl.ds(0, tail * EMBED)],
                out_hbm.at[pl.ds(v0 * EMBED, tail * EMBED)], o_sems[0],
            )

            @pl.when(wid == nw - 1)
            def _():
                ic_t.start()
                ic_t.wait()
                oc_t.start()
                oc_t.wait()

    return tr_kernel


@functools.lru_cache(maxsize=None)
def _build(hist, batch, vocab):
    info = plsc.get_sparse_core_info()
    nw = info.num_cores * info.num_subcores
    bq = 1024
    n_bq = batch // bq
    n_tasks_total = hist * n_bq
    n_tasks = n_tasks_total // nw

    mesh = plsc.VectorSubcoreMesh(core_axis_name="c", subcore_axis_name="s")

    scratch = [
        pltpu.VMEM((n_tasks, bq), jnp.int32),
        pltpu.VMEM((3, bq, EMBED), jnp.float32),
        pltpu.VMEM((2, EMBED, bq), jnp.float32),
    ] + [pltpu.SemaphoreType.DMA] * 6

    @functools.partial(
        pl.kernel,
        out_type=jax.ShapeDtypeStruct((hist, EMBED, batch), jnp.float32),
        mesh=mesh,
        scratch_types=scratch,
        compiler_params=pltpu.CompilerParams(
            use_tc_tiling_on_sc=False,
            needs_layout_passes=False,
            disable_bounds_checks=True,
        ),
    )
    def emb_kernel(idx_hbm, table_hbm, out_hbm, idxv, rows, outb, *sems):
        g_sems = sems[:3]
        o_sems = sems[3:5]
        i_sem = sems[5]
        wid = lax.axis_index("s") * info.num_cores + lax.axis_index("c")
        t_base = wid * n_tasks

        def task_hb(k):
            t = t_base + k
            return t // n_bq, (t % n_bq) * bq

        idx_cps = []
        for k in range(n_tasks):
            h, b0 = task_hb(k)
            idx_cps.append(pltpu.async_copy(
                idx_hbm.at[h, pl.ds(b0, bq)], idxv.at[k], i_sem
            ))
        for cp in idx_cps:
            cp.wait()

        def gather_start(k, s):
            return pltpu.async_copy(
                table_hbm.at[idxv.at[k]], rows.at[s], g_sems[s]
            )

        def out_start(k, s):
            h, b0 = task_hb(k)
            return pltpu.async_copy(
                outb.at[s], out_hbm.at[h, :, pl.ds(b0, bq)], o_sems[s]
            )

        lanes = lax.iota(jnp.int32, EMBED)
        col_idx = [jnp.full((EMBED,), e, jnp.int32) for e in range(EMBED)]

        def transpose(sr, so):
            @plsc.parallel_loop(0, bq // EMBED, unroll=4)
            def body(j):
                row_idx = lanes + j * EMBED
                for e in range(EMBED):
                    v = plsc.load_gather(rows.at[sr], [row_idx, col_idx[e]])
                    outb[so, e, pl.ds(j * EMBED, EMBED)] = v

        def finish(tp):
            srp, so = tp % 3, tp % 2
            g_cp[srp].wait()
            if o_cp[so] is not None:
                o_cp[so].wait()
            transpose(srp, so)
            o_cp[so] = out_start(tp, so)

        g_cp = [None, None, None]
        o_cp = [None, None]
        for t in range(n_tasks):
            g_cp[t % 3] = gather_start(t, t % 3)
            if t > 0:
                finish(t - 1)
        finish(n_tasks - 1)
        for s in range(2):
            if o_cp[s] is not None:
                o_cp[s].wait()

    return emb_kernel


def kernel(x, table):
    batch, _, hist = x.shape
    vocab = table.shape[0]
    idx_t = x[:, 0, :].T
    tail_flat = table[(vocab // 512) * 512:, :].reshape(-1)
    table_rm = _build_transpose(vocab)(table.T, tail_flat).reshape(vocab, EMBED)
    fn = _build(hist, batch, vocab)
    out = fn(idx_t, table_rm)
    return jnp.transpose(out, (2, 0, 1))
